# baseline (speedup 1.0000x reference)
import functools

import jax
import jax.numpy as jnp
from jax import lax
from jax.experimental import pallas as pl
from jax.experimental.pallas import tpu as pltpu
from jax.experimental.pallas import tpu_sc as plsc

N_NODES = 50000
D = 768
MSG_DIM = 100
MSG_PAD = 128
RAW_DIM = 3 * D
HID = RAW_DIM // 2
B = 8192
BE = 512


CW = 1024


def _dedup_body(si_ref, scat_ref, fix_ref):
    i = pl.program_id(0)
    src_e_col = si_ref[0, pl.ds(i * BE, BE)].reshape(BE, 1)
    row_ids = i * BE + lax.broadcasted_iota(jnp.int32, (BE, 1), 0)

    dup = jnp.zeros((BE, 1), jnp.bool_)
    for j in range(B // CW):
        cols = si_ref[0, pl.ds(j * CW, CW)].reshape(1, CW)
        col_ids = j * CW + lax.broadcasted_iota(jnp.int32, (1, CW), 1)
        hit = (src_e_col == cols) & (col_ids > row_ids)
        dup = dup | jnp.any(hit, axis=1, keepdims=True)
    rr = si_ref[0, B - 1]
    scat_ref[...] = jnp.where(dup, rr, src_e_col).reshape(1, BE)
    fix_ref[...] = jnp.full((1, 128), rr, jnp.int32)


def _dedup(src_idx):
    return pl.pallas_call(
        _dedup_body,
        grid=(B // BE,),
        in_specs=[pl.BlockSpec((1, B), lambda i: (0, 0))],
        out_specs=(pl.BlockSpec((1, BE), lambda i: (0, i)),
                   pl.BlockSpec((1, 128), lambda i: (0, 0))),
        out_shape=(jax.ShapeDtypeStruct((1, B), jnp.int32),
                   jax.ShapeDtypeStruct((1, 128), jnp.int32)),
    )(src_idx.reshape(1, B))


def _compute_body(dt_ref, ms_ref, md_ref, tw_ref, tb_ref, W1_ref, b1_ref,
                  W2_ref, b2_ref, Wx_ref, Wh_ref, bg_ref, out_ref):
    dt = dt_ref[...]
    ms = ms_ref[...]
    md = md_ref[...]
    y = dt * tw_ref[...] + tb_ref[...]
    rnd = (y + 12582912.0) - 12582912.0
    d = y - rnd
    u = d * d
    te = (0.99999944 + u * (-19.73903432 + u * (64.93061147 + u * (
        -85.29594601 + u * (58.91242234 + u * -21.28277633)))))
    W1 = W1_ref[...]
    f32 = jnp.float32
    bf16 = jnp.bfloat16
    msb = ms.astype(bf16)
    h1 = (jnp.dot(msb, W1[0:D], preferred_element_type=f32)
          + jnp.dot(md.astype(bf16), W1[D:2 * D], preferred_element_type=f32)
          + jnp.dot(te.astype(bf16), W1[2 * D:3 * D], preferred_element_type=f32)
          + b1_ref[...])
    h1 = jnp.maximum(h1, 0.0).astype(bf16)
    msg = jnp.dot(h1, W2_ref[...], preferred_element_type=f32) + b2_ref[...]
    gx = (jnp.dot(msg.astype(bf16), Wx_ref[...], preferred_element_type=f32)
          + bg_ref[...])
    gh = jnp.dot(msb, Wh_ref[...], preferred_element_type=f32)
    xr, xz, xn = gx[:, 0:D], gx[:, D:2 * D], gx[:, 2 * D:3 * D]
    hr, hz, hn = gh[:, 0:D], gh[:, D:2 * D], gh[:, 2 * D:3 * D]
    r = jax.nn.sigmoid(xr + hr)
    z = jax.nn.sigmoid(xz + hz)
    n = jnp.tanh(xn + r * hn)
    out_ref[...] = (1.0 - z) * n + z * ms


def _compute_h_new(dt, mem_src, mem_dst, tw, tb, W1, b1, W2p, b2p, Wxp, Wh,
                   bg):
    grid = (B // BE,)
    blk = lambda r, c: pl.BlockSpec((r, c), lambda i: (i, 0))
    full = lambda r, c: pl.BlockSpec((r, c), lambda i: (0, 0))
    return pl.pallas_call(
        _compute_body,
        grid=grid,
        in_specs=[
            blk(BE, 1),
            blk(BE, D),
            blk(BE, D),
            full(1, D),
            full(1, D),
            full(RAW_DIM, HID),
            full(1, HID),
            full(HID, MSG_PAD),
            full(1, MSG_PAD),
            full(MSG_PAD, 3 * D),
            full(D, 3 * D),
            full(1, 3 * D),
        ],
        out_specs=blk(BE, D),
        out_shape=jax.ShapeDtypeStruct((B, D), jnp.float32),
    )(dt, mem_src, mem_dst, tw.reshape(1, D), tb.reshape(1, D), W1,
      b1.reshape(1, HID), W2p, b2p, Wxp, Wh, bg.reshape(1, 3 * D))


NW = 32
EV_W = B // NW
GCH = 64


def _sc_gather(memory, src_idx, dst_idx, last_update):
    mesh = plsc.VectorSubcoreMesh(core_axis_name="c", subcore_axis_name="s")

    @functools.partial(
        pl.kernel,
        out_type=(
            jax.ShapeDtypeStruct((B, D), jnp.float32),
            jax.ShapeDtypeStruct((B, D), jnp.float32),
            jax.ShapeDtypeStruct((B,), jnp.float32),
        ),
        mesh=mesh,
        scratch_types=[
            pltpu.VMEM((GCH,), jnp.int32),
            pltpu.VMEM((GCH, D), jnp.float32),
            pltpu.VMEM((GCH,), jnp.float32),
            pltpu.SemaphoreType.DMA,
            pltpu.SemaphoreType.DMA,
        ],
    )
    def k(mem_hbm, src_hbm, dst_hbm, lu_hbm, osrc_hbm, odst_hbm, olu_hbm,
          idx_v, rows_v, lu_v, sem, sem2):
        c = lax.axis_index("c")
        s = lax.axis_index("s")
        wid = s * 2 + c
        base = wid * EV_W

        def do_rows(idx_hbm, out_hbm, with_lu):
            for ch in range(EV_W // GCH):
                off = base + ch * GCH
                pltpu.sync_copy(idx_hbm.at[pl.ds(off, GCH)], idx_v)
                pltpu.async_copy(mem_hbm.at[idx_v], rows_v, sem).wait()
                if with_lu:
                    pltpu.async_copy(lu_hbm.at[idx_v], lu_v, sem2).wait()
                    pltpu.sync_copy(lu_v, olu_hbm.at[pl.ds(off, GCH)])
                pltpu.sync_copy(rows_v, out_hbm.at[pl.ds(off, GCH)])

        do_rows(src_hbm, osrc_hbm, True)
        do_rows(dst_hbm, odst_hbm, False)

    return k(memory, src_idx, dst_idx, last_update)


SCW = 16
ROWS_W = 3128
CCH = 136
NCCH = ROWS_W // CCH
TAILR = (N_NODES // CCH) * CCH
SCH = 128


CCH2 = 80
CNCH = 19
CEXTRA = NW * CNCH * CCH2


def _sc_copy(tbl_ref, memory):
    mesh = plsc.VectorSubcoreMesh(core_axis_name="c", subcore_axis_name="s")

    @functools.partial(
        pl.kernel,
        out_type=(),
        mesh=mesh,
        scratch_types=[
            pltpu.VMEM((CCH2, D), jnp.float32),
            pltpu.VMEM((CCH2, D), jnp.float32),
            pltpu.SemaphoreType.DMA,
            pltpu.SemaphoreType.DMA,
            pltpu.SemaphoreType.DMA,
            pltpu.SemaphoreType.DMA,
        ],
    )
    def k(mem_hbm, tbl_hbm, cbuf0, cbuf1, semr0, semr1, semw0, semw1):
        wid = lax.axis_index("s") * 2 + lax.axis_index("c")
        r0 = wid * (CNCH * CCH2)
        bufs = (cbuf0, cbuf1)
        semr = (semr0, semr1)
        semw = (semw0, semw1)
        rd = [None] * CNCH
        wr = [None] * CNCH
        for ch in range(CNCH):
            p = ch % 2
            off = r0 + ch * CCH2
            if ch >= 2:
                wr[ch - 2].wait()
            rd[ch] = pltpu.async_copy(mem_hbm.at[pl.ds(off, CCH2)],
                                      bufs[p], semr[p])
            if ch >= 1:
                rd[ch - 1].wait()
                wr[ch - 1] = pltpu.async_copy(
                    bufs[1 - p], tbl_hbm.at[pl.ds(off - CCH2, CCH2)],
                    semw[1 - p])
        rd[CNCH - 1].wait()
        wr[CNCH - 1] = pltpu.async_copy(
            bufs[(CNCH - 1) % 2],
            tbl_hbm.at[pl.ds(r0 + (CNCH - 1) * CCH2, CCH2)],
            semw[(CNCH - 1) % 2])
        wr[CNCH - 2].wait()
        wr[CNCH - 1].wait()

        @pl.when(wid < (N_NODES - CEXTRA) // CCH2)
        def _cp_extra():
            off = CEXTRA + wid * CCH2
            pltpu.sync_copy(mem_hbm.at[pl.ds(off, CCH2)], cbuf0)
            pltpu.sync_copy(cbuf0, tbl_hbm.at[pl.ds(off, CCH2)])

    return k(memory, tbl_ref)


def _sc_scatter(tbl_ref, h_new, scat_idx):
    mesh = plsc.VectorSubcoreMesh(core_axis_name="c", subcore_axis_name="s")

    @functools.partial(
        pl.kernel,
        out_type=(),
        mesh=mesh,
        scratch_types=[
            pltpu.VMEM((SCH, D), jnp.float32),
            pltpu.VMEM((SCH,), jnp.int32),
            pltpu.SemaphoreType.DMA,
        ],
    )
    def k(h_hbm, si_hbm, tbl_hbm, rbuf, ibuf, sem):
        wid = lax.axis_index("s") * 2 + lax.axis_index("c")
        e0 = wid * (B // NW)
        for ch in range((B // NW) // SCH):
            off = e0 + ch * SCH
            pltpu.sync_copy(si_hbm.at[pl.ds(off, SCH)], ibuf)
            pltpu.sync_copy(h_hbm.at[pl.ds(off, SCH)], rbuf)
            pltpu.async_copy(rbuf, tbl_hbm.at[ibuf], sem).wait()

    return k(h_new, scat_idx, tbl_ref)


def _sc_fix(tbl_ref, h_new, fix_idx):
    mesh = plsc.VectorSubcoreMesh(core_axis_name="c", subcore_axis_name="s")

    @functools.partial(
        pl.kernel,
        out_type=(),
        mesh=mesh,
        scratch_types=[
            pltpu.VMEM((16, D), jnp.float32),
            pltpu.VMEM((16,), jnp.int32),
            pltpu.VMEM((16,), jnp.int32),
            pltpu.SemaphoreType.DMA,
        ],
    )
    def k(h_hbm, fx_hbm, tbl_hbm, fix_r, fix_i, fix_e, sem):
        wid = lax.axis_index("s") * 2 + lax.axis_index("c")

        @pl.when(wid == 0)
        def _fix():
            pltpu.sync_copy(fx_hbm.at[pl.ds(0, 16)], fix_i)
            fix_e[...] = jnp.full((16,), B - 1, jnp.int32)
            pltpu.async_copy(h_hbm.at[fix_e], fix_r, sem).wait()
            pltpu.async_copy(fix_r, tbl_hbm.at[fix_i], sem).wait()

    return k(h_new, fix_idx, tbl_ref)


def kernel(memory, last_update, edge_times, tw, tb, W1, b1, W2, b2, Wx, Wh,
           bg, src_idx, dst_idx):
    bf16 = jnp.bfloat16
    W2p = jnp.pad(W2, ((0, 0), (0, MSG_PAD - MSG_DIM))).astype(bf16)
    b2p = jnp.pad(b2, (0, MSG_PAD - MSG_DIM)).reshape(1, MSG_PAD)
    Wxp = jnp.pad(Wx, ((0, MSG_PAD - MSG_DIM), (0, 0))).astype(bf16)
    W1 = W1.astype(bf16)
    Wh = Wh.astype(bf16)
    inv2pi = 0.15915494309189535
    tw = tw * inv2pi
    tb = tb * inv2pi

    tbl = jax.new_ref(lax.empty((N_NODES, D), jnp.float32))
    _sc_copy(tbl, memory)
    scat_idx, fix_idx = _dedup(src_idx)
    mem_src, mem_dst, lu_src = _sc_gather(memory, src_idx, dst_idx,
                                          last_update)
    t = edge_times / 60.0
    dt = (t - lu_src).reshape(B, 1)
    h_new = _compute_h_new(dt, mem_src, mem_dst, tw, tb, W1, b1, W2p, b2p,
                           Wxp, Wh, bg)
    _sc_scatter(tbl, h_new, scat_idx.reshape(B))
    _sc_fix(tbl, h_new, fix_idx.reshape(128))
    return jax.freeze(tbl)


# speedup vs baseline: 1.0338x; 1.0338x over previous
import functools

import jax
import jax.numpy as jnp
from jax import lax
from jax.experimental import pallas as pl
from jax.experimental.pallas import tpu as pltpu
from jax.experimental.pallas import tpu_sc as plsc

N_NODES = 50000
D = 768
MSG_DIM = 100
MSG_PAD = 128
RAW_DIM = 3 * D
HID = RAW_DIM // 2
B = 8192
BE = 512


CW = 1024


def _dedup_body(si_ref, scat_ref, fix_ref):
    i = pl.program_id(0)
    src_e_col = si_ref[0, pl.ds(i * BE, BE)].reshape(BE, 1)
    row_ids = i * BE + lax.broadcasted_iota(jnp.int32, (BE, 1), 0)

    dup = jnp.zeros((BE, 1), jnp.bool_)
    for j in range(B // CW):
        cols = si_ref[0, pl.ds(j * CW, CW)].reshape(1, CW)
        col_ids = j * CW + lax.broadcasted_iota(jnp.int32, (1, CW), 1)
        hit = (src_e_col == cols) & (col_ids > row_ids)
        dup = dup | jnp.any(hit, axis=1, keepdims=True)
    rr = si_ref[0, B - 1]
    scat_ref[...] = jnp.where(dup, rr, src_e_col).reshape(1, BE)
    fix_ref[...] = jnp.full((1, 128), rr, jnp.int32)


def _dedup(src_idx):
    return pl.pallas_call(
        _dedup_body,
        grid=(B // BE,),
        in_specs=[pl.BlockSpec((1, B), lambda i: (0, 0))],
        out_specs=(pl.BlockSpec((1, BE), lambda i: (0, i)),
                   pl.BlockSpec((1, 128), lambda i: (0, 0))),
        out_shape=(jax.ShapeDtypeStruct((1, B), jnp.int32),
                   jax.ShapeDtypeStruct((1, 128), jnp.int32)),
    )(src_idx.reshape(1, B))


def _compute_body(dt_ref, ms_ref, md_ref, tw_ref, tb_ref, W1_ref, b1_ref,
                  W2_ref, b2_ref, Wx_ref, Wh_ref, bg_ref, out_ref):
    dt = dt_ref[...]
    ms = ms_ref[...]
    md = md_ref[...]
    y = dt * tw_ref[...] + tb_ref[...]
    rnd = (y + 12582912.0) - 12582912.0
    d = y - rnd
    u = d * d
    te = (0.99999944 + u * (-19.73903432 + u * (64.93061147 + u * (
        -85.29594601 + u * (58.91242234 + u * -21.28277633)))))
    W1 = W1_ref[...]
    f32 = jnp.float32
    bf16 = jnp.bfloat16
    msb = ms.astype(bf16)
    h1 = (jnp.dot(msb, W1[0:D], preferred_element_type=f32)
          + jnp.dot(md.astype(bf16), W1[D:2 * D], preferred_element_type=f32)
          + jnp.dot(te.astype(bf16), W1[2 * D:3 * D], preferred_element_type=f32)
          + b1_ref[...])
    h1 = jnp.maximum(h1, 0.0).astype(bf16)
    msg = jnp.dot(h1, W2_ref[...], preferred_element_type=f32) + b2_ref[...]
    gx = (jnp.dot(msg.astype(bf16), Wx_ref[...], preferred_element_type=f32)
          + bg_ref[...])
    gh = jnp.dot(msb, Wh_ref[...], preferred_element_type=f32)
    xr, xz, xn = gx[:, 0:D], gx[:, D:2 * D], gx[:, 2 * D:3 * D]
    hr, hz, hn = gh[:, 0:D], gh[:, D:2 * D], gh[:, 2 * D:3 * D]
    r = jax.nn.sigmoid(xr + hr)
    z = jax.nn.sigmoid(xz + hz)
    n = jnp.tanh(xn + r * hn)
    out_ref[...] = (1.0 - z) * n + z * ms


def _compute_h_new(dt, mem_src, mem_dst, tw, tb, W1, b1, W2p, b2p, Wxp, Wh,
                   bg):
    grid = (B // BE,)
    blk = lambda r, c: pl.BlockSpec((r, c), lambda i: (i, 0))
    full = lambda r, c: pl.BlockSpec((r, c), lambda i: (0, 0))
    return pl.pallas_call(
        _compute_body,
        grid=grid,
        in_specs=[
            blk(BE, 1),
            blk(BE, D),
            blk(BE, D),
            full(1, D),
            full(1, D),
            full(RAW_DIM, HID),
            full(1, HID),
            full(HID, MSG_PAD),
            full(1, MSG_PAD),
            full(MSG_PAD, 3 * D),
            full(D, 3 * D),
            full(1, 3 * D),
        ],
        out_specs=blk(BE, D),
        out_shape=jax.ShapeDtypeStruct((B, D), jnp.float32),
    )(dt, mem_src, mem_dst, tw.reshape(1, D), tb.reshape(1, D), W1,
      b1.reshape(1, HID), W2p, b2p, Wxp, Wh, bg.reshape(1, 3 * D))


NW = 32
EV_W = B // NW
GCH = 64


def _sc_gather(memory, src_idx, dst_idx, last_update):
    mesh = plsc.VectorSubcoreMesh(core_axis_name="c", subcore_axis_name="s")

    @functools.partial(
        pl.kernel,
        out_type=(
            jax.ShapeDtypeStruct((B, D), jnp.float32),
            jax.ShapeDtypeStruct((B, D), jnp.float32),
            jax.ShapeDtypeStruct((B,), jnp.float32),
        ),
        mesh=mesh,
        scratch_types=[
            pltpu.VMEM((GCH,), jnp.int32),
            pltpu.VMEM((GCH, D), jnp.float32),
            pltpu.VMEM((GCH,), jnp.float32),
            pltpu.SemaphoreType.DMA,
            pltpu.SemaphoreType.DMA,
        ],
    )
    def k(mem_hbm, src_hbm, dst_hbm, lu_hbm, osrc_hbm, odst_hbm, olu_hbm,
          idx_v, rows_v, lu_v, sem, sem2):
        c = lax.axis_index("c")
        s = lax.axis_index("s")
        wid = s * 2 + c
        base = wid * EV_W

        def do_rows(idx_hbm, out_hbm, with_lu):
            for ch in range(EV_W // GCH):
                off = base + ch * GCH
                pltpu.sync_copy(idx_hbm.at[pl.ds(off, GCH)], idx_v)
                pltpu.async_copy(mem_hbm.at[idx_v], rows_v, sem).wait()
                if with_lu:
                    pltpu.async_copy(lu_hbm.at[idx_v], lu_v, sem2).wait()
                    pltpu.sync_copy(lu_v, olu_hbm.at[pl.ds(off, GCH)])
                pltpu.sync_copy(rows_v, out_hbm.at[pl.ds(off, GCH)])

        do_rows(src_hbm, osrc_hbm, True)
        do_rows(dst_hbm, odst_hbm, False)

    return k(memory, src_idx, dst_idx, last_update)


SCW = 16
ROWS_W = 3128
CCH = 136
NCCH = ROWS_W // CCH
TAILR = (N_NODES // CCH) * CCH
SCH = 128


CCH2 = 80
CNCH = 19
CEXTRA = NW * CNCH * CCH2


def _sc_copy(tbl_ref, memory):
    mesh = plsc.VectorSubcoreMesh(core_axis_name="c", subcore_axis_name="s")

    @functools.partial(
        pl.kernel,
        out_type=(),
        mesh=mesh,
        scratch_types=[
            pltpu.VMEM((CCH2, D), jnp.float32),
            pltpu.VMEM((CCH2, D), jnp.float32),
            pltpu.SemaphoreType.DMA,
            pltpu.SemaphoreType.DMA,
            pltpu.SemaphoreType.DMA,
            pltpu.SemaphoreType.DMA,
        ],
    )
    def k(mem_hbm, tbl_hbm, cbuf0, cbuf1, semr0, semr1, semw0, semw1):
        wid = lax.axis_index("s") * 2 + lax.axis_index("c")
        r0 = wid * (CNCH * CCH2)
        for ch in range(CNCH):
            off = r0 + ch * CCH2
            pltpu.sync_copy(mem_hbm.at[pl.ds(off, CCH2)], cbuf0)
            pltpu.sync_copy(cbuf0, tbl_hbm.at[pl.ds(off, CCH2)])

        @pl.when(wid < (N_NODES - CEXTRA) // CCH2)
        def _cp_extra():
            off = CEXTRA + wid * CCH2
            pltpu.sync_copy(mem_hbm.at[pl.ds(off, CCH2)], cbuf0)
            pltpu.sync_copy(cbuf0, tbl_hbm.at[pl.ds(off, CCH2)])

    return k(memory, tbl_ref)


def _sc_scatter(tbl_ref, h_new, scat_idx):
    mesh = plsc.VectorSubcoreMesh(core_axis_name="c", subcore_axis_name="s")

    @functools.partial(
        pl.kernel,
        out_type=(),
        mesh=mesh,
        scratch_types=[
            pltpu.VMEM((SCH, D), jnp.float32),
            pltpu.VMEM((SCH,), jnp.int32),
            pltpu.SemaphoreType.DMA,
        ],
    )
    def k(h_hbm, si_hbm, tbl_hbm, rbuf, ibuf, sem):
        wid = lax.axis_index("s") * 2 + lax.axis_index("c")
        e0 = wid * (B // NW)
        for ch in range((B // NW) // SCH):
            off = e0 + ch * SCH
            pltpu.sync_copy(si_hbm.at[pl.ds(off, SCH)], ibuf)
            pltpu.sync_copy(h_hbm.at[pl.ds(off, SCH)], rbuf)
            pltpu.async_copy(rbuf, tbl_hbm.at[ibuf], sem).wait()

    return k(h_new, scat_idx, tbl_ref)


def _sc_fix(tbl_ref, h_new, fix_idx):
    mesh = plsc.VectorSubcoreMesh(core_axis_name="c", subcore_axis_name="s")

    @functools.partial(
        pl.kernel,
        out_type=(),
        mesh=mesh,
        scratch_types=[
            pltpu.VMEM((16, D), jnp.float32),
            pltpu.VMEM((16,), jnp.int32),
            pltpu.VMEM((16,), jnp.int32),
            pltpu.SemaphoreType.DMA,
        ],
    )
    def k(h_hbm, fx_hbm, tbl_hbm, fix_r, fix_i, fix_e, sem):
        wid = lax.axis_index("s") * 2 + lax.axis_index("c")

        @pl.when(wid == 0)
        def _fix():
            pltpu.sync_copy(fx_hbm.at[pl.ds(0, 16)], fix_i)
            fix_e[...] = jnp.full((16,), B - 1, jnp.int32)
            pltpu.async_copy(h_hbm.at[fix_e], fix_r, sem).wait()
            pltpu.async_copy(fix_r, tbl_hbm.at[fix_i], sem).wait()

    return k(h_new, fix_idx, tbl_ref)


def kernel(memory, last_update, edge_times, tw, tb, W1, b1, W2, b2, Wx, Wh,
           bg, src_idx, dst_idx):
    bf16 = jnp.bfloat16
    W2p = jnp.pad(W2, ((0, 0), (0, MSG_PAD - MSG_DIM))).astype(bf16)
    b2p = jnp.pad(b2, (0, MSG_PAD - MSG_DIM)).reshape(1, MSG_PAD)
    Wxp = jnp.pad(Wx, ((0, MSG_PAD - MSG_DIM), (0, 0))).astype(bf16)
    W1 = W1.astype(bf16)
    Wh = Wh.astype(bf16)
    inv2pi = 0.15915494309189535
    tw = tw * inv2pi
    tb = tb * inv2pi

    tbl = jax.new_ref(lax.empty((N_NODES, D), jnp.float32))
    _sc_copy(tbl, memory)
    scat_idx, fix_idx = _dedup(src_idx)
    mem_src, mem_dst, lu_src = _sc_gather(memory, src_idx, dst_idx,
                                          last_update)
    t = edge_times / 60.0
    dt = (t - lu_src).reshape(B, 1)
    h_new = _compute_h_new(dt, mem_src, mem_dst, tw, tb, W1, b1, W2p, b2p,
                           Wxp, Wh, bg)
    _sc_scatter(tbl, h_new, scat_idx.reshape(B))
    _sc_fix(tbl, h_new, fix_idx.reshape(128))
    return jax.freeze(tbl)
